# baseline (speedup 1.0000x reference)
import functools

import jax
import jax.numpy as jnp
from jax import lax
from jax.experimental import pallas as pl
from jax.experimental.pallas import tpu as pltpu
from jax.experimental.pallas import tpu_sc as plsc

DIM = 1024
D2 = DIM // 2
NTOK = 4 * 8192


_IDX_R, _IDX_C = 256, 128


def _table_index_body(tt_ref, tv_ref, np_ref, emb_ref, p8_ref, bigT_ref, g_ref):
    p8 = p8_ref[...]

    bigT_ref[:, :, D2:] = jnp.broadcast_to(p8[None, :, :], (24, 8, D2))
    bigT_ref[0:8, :, 0:D2] = jnp.broadcast_to(p8[:, None, :], (8, 8, D2))
    bigT_ref[8:24, :, 0:D2] = jnp.broadcast_to(emb_ref[...][:, None, :],
                                               (16, 8, D2))

    tt = tt_ref[...]
    tv = tv_ref[...]
    npos = np_ref[...]
    present = [jnp.sum(jnp.where(npos == v, 1, 0)) > 0 for v in range(6)]
    db = jnp.full((_IDX_R, _IDX_C), -1, jnp.int32)
    for v in range(5, -1, -1):
        db = jnp.where(jnp.logical_and(present[v], tv <= v), v, db)
    mp = jnp.int32(-1)
    for v in range(6):
        mp = jnp.where(present[v], jnp.int32(v), mp)
    db = jnp.where(db >= 0, db, mp)

    c = jnp.full((_IDX_R, _IDX_C), 19, jnp.int32)
    c = jnp.where(tt == 0, 8, c)
    c = jnp.where(tt == 1, 9 + tv, c)
    c = jnp.where(tt == 2, 13 + tv, c)
    c = jnp.where(tt == 4, db, c)
    g_ref[...] = c * 8 + npos


def _build_table_and_index(tt, tv, npos, emb16, p8):
    return pl.pallas_call(
        _table_index_body,
        out_shape=[
            jax.ShapeDtypeStruct((24, 8, DIM), jnp.float32),
            jax.ShapeDtypeStruct((_IDX_R, _IDX_C), jnp.int32),
        ],
    )(tt, tv, npos, emb16, p8)


_NC = 2
_NS = 16
_NW = _NC * _NS
_BPW = NTOK // _NW
_K = 32
_NCHUNK = _BPW // _K


def _gather_body(table_hbm, idx_hbm, out_hbm,
                 idx_v, buf0, buf1, sg0, sg1, sw0, sw1):
    wid = lax.axis_index("s") * _NC + lax.axis_index("c")
    base = wid * _BPW
    pltpu.sync_copy(idx_hbm.at[wid], idx_v)

    def step(t, carry):
        c0 = 2 * t
        c1 = c0 + 1

        @pl.when(t > 0)
        def _():
            pltpu.make_async_copy(buf0, out_hbm.at[pl.ds(base, _K)], sw0).wait()

        g0 = pltpu.async_copy(table_hbm.at[idx_v.at[c0]], buf0, sg0)

        @pl.when(t > 0)
        def _():
            pltpu.make_async_copy(buf1, out_hbm.at[pl.ds(base, _K)], sw1).wait()

        g1 = pltpu.async_copy(table_hbm.at[idx_v.at[c1]], buf1, sg1)

        g0.wait()
        pltpu.async_copy(buf0, out_hbm.at[pl.ds(base + c0 * _K, _K)], sw0)
        g1.wait()
        pltpu.async_copy(buf1, out_hbm.at[pl.ds(base + c1 * _K, _K)], sw1)
        return carry

    lax.fori_loop(0, _NCHUNK // 2, step, 0, unroll=False)
    pltpu.make_async_copy(buf0, out_hbm.at[pl.ds(base, _K)], sw0).wait()
    pltpu.make_async_copy(buf1, out_hbm.at[pl.ds(base, _K)], sw1).wait()


@functools.cache
def _gather_rows_kernel():
    return functools.partial(
        pl.kernel,
        mesh=plsc.VectorSubcoreMesh(core_axis_name="c", subcore_axis_name="s"),
        out_type=jax.ShapeDtypeStruct((NTOK, DIM), jnp.float32),
        scratch_types=[
            pltpu.VMEM((_NCHUNK, _K), jnp.int32),
            pltpu.VMEM((_K, DIM), jnp.float32),
            pltpu.VMEM((_K, DIM), jnp.float32),
            pltpu.SemaphoreType.DMA,
            pltpu.SemaphoreType.DMA,
            pltpu.SemaphoreType.DMA,
            pltpu.SemaphoreType.DMA,
        ],
    )(_gather_body)


def _lu_tri_solve(lu, b, unit_lower_then_upper=True):
    y = lax.linalg.triangular_solve(lu, b, left_side=True, lower=True,
                                    unit_diagonal=True)
    return lax.linalg.triangular_solve(lu, y, left_side=True, lower=False)


def _prepare(dense_batch, embeddings, primitives_raw, identity):
    f32 = jnp.float32
    X = jnp.tril(primitives_raw.astype(f32))
    A = X - jnp.swapaxes(X, -1, -2)
    I = jnp.eye(D2, dtype=f32)
    B = I - 0.5 * A
    C = I + 0.5 * A
    lu, _, perm = lax.linalg.lu(B)

    v0 = identity.astype(f32).reshape(D2)
    r1 = jnp.einsum("bij,j->bi", C, v0)
    r1p = jnp.take_along_axis(r1, perm, axis=1)[..., None]
    s1 = _lu_tri_solve(lu, r1p)
    e2 = s1[0, :, 0]
    e3 = s1[1, :, 0]
    r2 = C[0] @ jnp.stack([e2, e3], axis=1)
    r2p = r2[perm[0], :]
    s2 = _lu_tri_solve(lu[0], r2p)

    zero = jnp.zeros((D2,), f32)
    p8 = jnp.stack([v0, v0, e2, e3, s2[:, 0], s2[:, 1], zero, zero])

    emb16 = jnp.pad(embeddings.astype(f32), ((0, 5), (0, 0)))
    tt = dense_batch[0].reshape(_IDX_R, _IDX_C)
    tv = dense_batch[1].reshape(_IDX_R, _IDX_C)
    npos = dense_batch[2].reshape(_IDX_R, _IDX_C)
    return tt, tv, npos, emb16, p8


def kernel(dense_batch, embeddings, primitives_raw, identity):
    tt, tv, npos, emb16, p8 = _prepare(dense_batch, embeddings,
                                       primitives_raw, identity)
    bigT3, g = _build_table_and_index(tt, tv, npos, emb16, p8)
    bigT = bigT3.reshape(24 * 8, DIM)
    gidx = g.reshape(_NW, _NCHUNK, _K)

    out = _gather_rows_kernel()(bigT, gidx)
    return out.reshape(4, 8192, DIM)


# speedup vs baseline: 1.7741x; 1.7741x over previous
import functools

import jax
import jax.numpy as jnp
from jax import lax
from jax.experimental import pallas as pl
from jax.experimental.pallas import tpu as pltpu
from jax.experimental.pallas import tpu_sc as plsc

DIM = 1024
D2 = DIM // 2
NTOK = 4 * 8192


_IDX_R, _IDX_C = 256, 128


def _table_index_body(tt_ref, tv_ref, np_ref, emb_ref, primT_ref, id_ref,
                      bigT_ref, g_ref):
    id8 = jnp.broadcast_to(id_ref[...], (8, D2))
    p0t = primT_ref[0]
    p1t = primT_ref[1]
    x1 = jnp.dot(id8, p0t, preferred_element_type=jnp.float32)
    y1 = jnp.dot(id8, p1t, preferred_element_type=jnp.float32)
    x2 = jnp.dot(x1, p0t, preferred_element_type=jnp.float32)
    y2 = jnp.dot(y1, p0t, preferred_element_type=jnp.float32)
    rid = lax.broadcasted_iota(jnp.int32, (8, D2), 0)
    p8 = jnp.where(rid < 2, id8,
         jnp.where(rid == 2, x1,
         jnp.where(rid == 3, y1,
         jnp.where(rid == 4, x2,
         jnp.where(rid == 5, y2, jnp.zeros_like(id8))))))

    bigT_ref[:, :, D2:] = jnp.broadcast_to(p8[None, :, :], (24, 8, D2))
    bigT_ref[0:8, :, 0:D2] = jnp.broadcast_to(p8[:, None, :], (8, 8, D2))
    bigT_ref[8:24, :, 0:D2] = jnp.broadcast_to(emb_ref[...][:, None, :],
                                               (16, 8, D2))

    tt = tt_ref[...]
    tv = tv_ref[...]
    npos = np_ref[...]
    present = [jnp.sum(jnp.where(npos == v, 1, 0)) > 0 for v in range(6)]
    db = jnp.full((_IDX_R, _IDX_C), -1, jnp.int32)
    for v in range(5, -1, -1):
        db = jnp.where(jnp.logical_and(present[v], tv <= v), v, db)
    mp = jnp.int32(-1)
    for v in range(6):
        mp = jnp.where(present[v], jnp.int32(v), mp)
    db = jnp.where(db >= 0, db, mp)

    c = jnp.full((_IDX_R, _IDX_C), 19, jnp.int32)
    c = jnp.where(tt == 0, 8, c)
    c = jnp.where(tt == 1, 9 + tv, c)
    c = jnp.where(tt == 2, 13 + tv, c)
    c = jnp.where(tt == 4, db, c)
    g_ref[...] = c * 8 + npos


def _build_table_and_index(tt, tv, npos, emb16, primT, identity):
    return pl.pallas_call(
        _table_index_body,
        out_shape=[
            jax.ShapeDtypeStruct((24, 8, DIM), jnp.float32),
            jax.ShapeDtypeStruct((_IDX_R, _IDX_C), jnp.int32),
        ],
    )(tt, tv, npos, emb16, primT, identity)


def _table_index_body_p8(tt_ref, tv_ref, np_ref, emb_ref, p8_ref,
                         bigT_ref, g_ref):
    p8 = p8_ref[...]
    bigT_ref[:, :, D2:] = jnp.broadcast_to(p8[None, :, :], (24, 8, D2))
    bigT_ref[0:8, :, 0:D2] = jnp.broadcast_to(p8[:, None, :], (8, 8, D2))
    bigT_ref[8:24, :, 0:D2] = jnp.broadcast_to(emb_ref[...][:, None, :],
                                               (16, 8, D2))
    tt = tt_ref[...]
    tv = tv_ref[...]
    npos = np_ref[...]
    present = [jnp.sum(jnp.where(npos == v, 1, 0)) > 0 for v in range(6)]
    db = jnp.full((_IDX_R, _IDX_C), -1, jnp.int32)
    for v in range(5, -1, -1):
        db = jnp.where(jnp.logical_and(present[v], tv <= v), v, db)
    mp = jnp.int32(-1)
    for v in range(6):
        mp = jnp.where(present[v], jnp.int32(v), mp)
    db = jnp.where(db >= 0, db, mp)
    c = jnp.full((_IDX_R, _IDX_C), 19, jnp.int32)
    c = jnp.where(tt == 0, 8, c)
    c = jnp.where(tt == 1, 9 + tv, c)
    c = jnp.where(tt == 2, 13 + tv, c)
    c = jnp.where(tt == 4, db, c)
    g_ref[...] = c * 8 + npos


def _build_table_and_index_p8(tt, tv, npos, emb16, p8):
    return pl.pallas_call(
        _table_index_body_p8,
        out_shape=[
            jax.ShapeDtypeStruct((24, 8, DIM), jnp.float32),
            jax.ShapeDtypeStruct((_IDX_R, _IDX_C), jnp.int32),
        ],
    )(tt, tv, npos, emb16, p8)


_CG_IT1 = 220
_CG_IT2 = 220


def _cg_rows(Nmat, nrhs, iters):
    def matvec(p):
        t = jnp.dot(p, Nmat, preferred_element_type=jnp.float32)
        return p - jnp.dot(t, Nmat, preferred_element_type=jnp.float32)

    x0 = jnp.zeros_like(nrhs)
    rs0 = jnp.sum(nrhs * nrhs, axis=1, keepdims=True)

    def it(_, carry):
        x, r, p, rs = carry
        q = matvec(p)
        alpha = rs / jnp.sum(p * q, axis=1, keepdims=True)
        x = x + alpha * p
        r = r - alpha * q
        rs2 = jnp.sum(r * r, axis=1, keepdims=True)
        p = r + (rs2 / rs) * p
        return x, r, p, rs2

    x, _, _, _ = lax.fori_loop(0, iters, it, (x0, nrhs, nrhs, rs0))
    return x


def _cg_body(blkN_ref, vv_ref, v_ref, p8_ref):
    Nb = blkN_ref[...]
    N0 = blkN_ref[0:D2, 0:D2]
    vv8 = jnp.broadcast_to(vv_ref[...], (8, DIM))

    b1 = vv8 - jnp.dot(vv8, Nb, preferred_element_type=jnp.float32)
    n1 = b1 - jnp.dot(b1, Nb, preferred_element_type=jnp.float32)
    x1 = _cg_rows(Nb, n1, _CG_IT1)
    e2 = x1[0:1, 0:D2]
    e3 = x1[0:1, D2:DIM]

    rid = lax.broadcasted_iota(jnp.int32, (8, D2), 0)
    S = jnp.where(rid % 2 == 0, jnp.broadcast_to(e2, (8, D2)),
                  jnp.broadcast_to(e3, (8, D2)))
    b2 = S - jnp.dot(S, N0, preferred_element_type=jnp.float32)
    n2 = b2 - jnp.dot(b2, N0, preferred_element_type=jnp.float32)
    x2 = _cg_rows(N0, n2, _CG_IT2)

    vb = jnp.broadcast_to(v_ref[...], (8, D2))
    p8_ref[...] = jnp.where(rid < 2, vb,
                  jnp.where(rid == 2, jnp.broadcast_to(e2, (8, D2)),
                  jnp.where(rid == 3, jnp.broadcast_to(e3, (8, D2)),
                  jnp.where(rid < 6, x2, jnp.zeros((8, D2), jnp.float32)))))


def _cg_p8(blkN, vv, v):
    return pl.pallas_call(
        _cg_body,
        out_shape=jax.ShapeDtypeStruct((8, D2), jnp.float32),
    )(blkN, vv, v)


_NC = 2
_NS = 16
_NW = _NC * _NS
_BPW = NTOK // _NW
_K = 64
_NCHUNK = _BPW // _K


def _gather_body(table_hbm, idx_hbm, out_hbm, idx_v, rows_v, sem):
    wid = lax.axis_index("s") * _NC + lax.axis_index("c")
    base = wid * _BPW
    pltpu.sync_copy(idx_hbm.at[wid], idx_v)
    for ck in range(_NCHUNK):
        pltpu.async_copy(table_hbm.at[idx_v.at[ck]], rows_v, sem).wait()
        pltpu.sync_copy(rows_v, out_hbm.at[pl.ds(base + ck * _K, _K)])


@functools.cache
def _gather_rows_kernel():
    return functools.partial(
        pl.kernel,
        mesh=plsc.VectorSubcoreMesh(core_axis_name="c", subcore_axis_name="s"),
        out_type=jax.ShapeDtypeStruct((NTOK, DIM), jnp.float32),
        scratch_types=[
            pltpu.VMEM((_NCHUNK, _K), jnp.int32),
            pltpu.VMEM((_K, DIM), jnp.float32),
            pltpu.SemaphoreType.DMA,
        ],
    )(_gather_body)


def _prepare(dense_batch, embeddings, primitives_raw, identity):
    f32 = jnp.float32
    X = jnp.tril(primitives_raw.astype(f32))
    A = X - jnp.swapaxes(X, -1, -2)
    N = 0.5 * A
    zblk = jnp.zeros((D2, D2), f32)
    blkN = jnp.block([[N[0], zblk], [zblk, N[1]]])

    v = identity.astype(f32).reshape(1, D2)
    vv = jnp.concatenate([v, v], axis=1)

    emb16 = jnp.pad(embeddings.astype(f32), ((0, 5), (0, 0)))
    tt = dense_batch[0].reshape(_IDX_R, _IDX_C)
    tv = dense_batch[1].reshape(_IDX_R, _IDX_C)
    npos = dense_batch[2].reshape(_IDX_R, _IDX_C)
    return tt, tv, npos, emb16, blkN, vv, v


def kernel(dense_batch, embeddings, primitives_raw, identity):
    tt, tv, npos, emb16, blkN, vv, v = _prepare(dense_batch, embeddings,
                                                primitives_raw, identity)
    p8 = _cg_p8(blkN, vv, v)
    bigT3, g = _build_table_and_index_p8(tt, tv, npos, emb16, p8)
    bigT = bigT3.reshape(24 * 8, DIM)
    gidx = g.reshape(_NW, _NCHUNK, _K)

    out = _gather_rows_kernel()(bigT, gidx)
    return out.reshape(4, 8192, DIM)


# speedup vs baseline: 2.3157x; 1.3052x over previous
import functools

import jax
import jax.numpy as jnp
from jax import lax
from jax.experimental import pallas as pl
from jax.experimental.pallas import tpu as pltpu
from jax.experimental.pallas import tpu_sc as plsc

DIM = 1024
D2 = DIM // 2
NTOK = 4 * 8192


_IDX_R, _IDX_C = 256, 128


def _table_index_body(tt_ref, tv_ref, np_ref, emb_ref, primT_ref, id_ref,
                      bigT_ref, g_ref):
    id8 = jnp.broadcast_to(id_ref[...], (8, D2))
    p0t = primT_ref[0]
    p1t = primT_ref[1]
    x1 = jnp.dot(id8, p0t, preferred_element_type=jnp.float32)
    y1 = jnp.dot(id8, p1t, preferred_element_type=jnp.float32)
    x2 = jnp.dot(x1, p0t, preferred_element_type=jnp.float32)
    y2 = jnp.dot(y1, p0t, preferred_element_type=jnp.float32)
    rid = lax.broadcasted_iota(jnp.int32, (8, D2), 0)
    p8 = jnp.where(rid < 2, id8,
         jnp.where(rid == 2, x1,
         jnp.where(rid == 3, y1,
         jnp.where(rid == 4, x2,
         jnp.where(rid == 5, y2, jnp.zeros_like(id8))))))

    bigT_ref[:, :, D2:] = jnp.broadcast_to(p8[None, :, :], (24, 8, D2))
    bigT_ref[0:8, :, 0:D2] = jnp.broadcast_to(p8[:, None, :], (8, 8, D2))
    bigT_ref[8:24, :, 0:D2] = jnp.broadcast_to(emb_ref[...][:, None, :],
                                               (16, 8, D2))

    tt = tt_ref[...]
    tv = tv_ref[...]
    npos = np_ref[...]
    present = [jnp.sum(jnp.where(npos == v, 1, 0)) > 0 for v in range(6)]
    db = jnp.full((_IDX_R, _IDX_C), -1, jnp.int32)
    for v in range(5, -1, -1):
        db = jnp.where(jnp.logical_and(present[v], tv <= v), v, db)
    mp = jnp.int32(-1)
    for v in range(6):
        mp = jnp.where(present[v], jnp.int32(v), mp)
    db = jnp.where(db >= 0, db, mp)

    c = jnp.full((_IDX_R, _IDX_C), 19, jnp.int32)
    c = jnp.where(tt == 0, 8, c)
    c = jnp.where(tt == 1, 9 + tv, c)
    c = jnp.where(tt == 2, 13 + tv, c)
    c = jnp.where(tt == 4, db, c)
    g_ref[...] = c * 8 + npos


def _build_table_and_index(tt, tv, npos, emb16, primT, identity):
    return pl.pallas_call(
        _table_index_body,
        out_shape=[
            jax.ShapeDtypeStruct((24, 8, DIM), jnp.float32),
            jax.ShapeDtypeStruct((_IDX_R, _IDX_C), jnp.int32),
        ],
    )(tt, tv, npos, emb16, primT, identity)


def _table_index_body_p8(tt_ref, tv_ref, np_ref, emb_ref, p8_ref,
                         bigT_ref, g_ref):
    p8 = p8_ref[...]
    bigT_ref[:, :, D2:] = jnp.broadcast_to(p8[None, :, :], (24, 8, D2))
    bigT_ref[0:8, :, 0:D2] = jnp.broadcast_to(p8[:, None, :], (8, 8, D2))
    bigT_ref[8:24, :, 0:D2] = jnp.broadcast_to(emb_ref[...][:, None, :],
                                               (16, 8, D2))
    tt = tt_ref[...]
    tv = tv_ref[...]
    npos = np_ref[...]
    present = [jnp.sum(jnp.where(npos == v, 1, 0)) > 0 for v in range(6)]
    db = jnp.full((_IDX_R, _IDX_C), -1, jnp.int32)
    for v in range(5, -1, -1):
        db = jnp.where(jnp.logical_and(present[v], tv <= v), v, db)
    mp = jnp.int32(-1)
    for v in range(6):
        mp = jnp.where(present[v], jnp.int32(v), mp)
    db = jnp.where(db >= 0, db, mp)
    c = jnp.full((_IDX_R, _IDX_C), 19, jnp.int32)
    c = jnp.where(tt == 0, 8, c)
    c = jnp.where(tt == 1, 9 + tv, c)
    c = jnp.where(tt == 2, 13 + tv, c)
    c = jnp.where(tt == 4, db, c)
    g_ref[...] = c * 8 + npos


def _build_table_and_index_p8(tt, tv, npos, emb16, p8):
    return pl.pallas_call(
        _table_index_body_p8,
        out_shape=[
            jax.ShapeDtypeStruct((24, 8, DIM), jnp.float32),
            jax.ShapeDtypeStruct((_IDX_R, _IDX_C), jnp.int32),
        ],
    )(tt, tv, npos, emb16, p8)


def _fused_body(tt_ref, tv_ref, np_ref, emb_ref, blkN_ref, vv_ref, v_ref,
                bigT_ref, g_ref):
    _cg_into(blkN_ref, vv_ref, v_ref, bigT_ref, emb_ref)
    _index_into(tt_ref, tv_ref, np_ref, g_ref)


def _cg_into(blkN_ref, vv_ref, v_ref, bigT_ref, emb_ref):
    Nb = blkN_ref[...]
    N0 = blkN_ref[0:D2, 0:D2]
    vv8 = jnp.broadcast_to(vv_ref[...], (8, DIM))
    b1 = vv8 - jnp.dot(vv8, Nb, preferred_element_type=jnp.float32)
    n1 = b1 - jnp.dot(b1, Nb, preferred_element_type=jnp.float32)
    x1 = _cg_rows(Nb, n1, _CG_IT1)
    e2 = x1[0:1, 0:D2]
    e3 = x1[0:1, D2:DIM]
    rid = lax.broadcasted_iota(jnp.int32, (8, D2), 0)
    S = jnp.where(rid % 2 == 0, jnp.broadcast_to(e2, (8, D2)),
                  jnp.broadcast_to(e3, (8, D2)))
    b2 = S - jnp.dot(S, N0, preferred_element_type=jnp.float32)
    n2 = b2 - jnp.dot(b2, N0, preferred_element_type=jnp.float32)
    x2 = _cg_rows(N0, n2, _CG_IT2)
    vb = jnp.broadcast_to(v_ref[...], (8, D2))
    p8 = jnp.where(rid < 2, vb,
         jnp.where(rid == 2, jnp.broadcast_to(e2, (8, D2)),
         jnp.where(rid == 3, jnp.broadcast_to(e3, (8, D2)),
         jnp.where(rid < 6, x2, jnp.zeros((8, D2), jnp.float32)))))
    bigT_ref[:, :, D2:] = jnp.broadcast_to(p8[None, :, :], (24, 8, D2))
    bigT_ref[0:8, :, 0:D2] = jnp.broadcast_to(p8[:, None, :], (8, 8, D2))
    bigT_ref[8:24, :, 0:D2] = jnp.broadcast_to(emb_ref[...][:, None, :],
                                               (16, 8, D2))


def _index_into(tt_ref, tv_ref, np_ref, g_ref):
    tt = tt_ref[...]
    tv = tv_ref[...]
    npos = np_ref[...]
    present = [jnp.sum(jnp.where(npos == v, 1, 0)) > 0 for v in range(6)]
    db = jnp.full((_IDX_R, _IDX_C), -1, jnp.int32)
    for v in range(5, -1, -1):
        db = jnp.where(jnp.logical_and(present[v], tv <= v), v, db)
    mp = jnp.int32(-1)
    for v in range(6):
        mp = jnp.where(present[v], jnp.int32(v), mp)
    db = jnp.where(db >= 0, db, mp)
    c = jnp.full((_IDX_R, _IDX_C), 19, jnp.int32)
    c = jnp.where(tt == 0, 8, c)
    c = jnp.where(tt == 1, 9 + tv, c)
    c = jnp.where(tt == 2, 13 + tv, c)
    c = jnp.where(tt == 4, db, c)
    g_ref[...] = c * 8 + npos


def _fused_table_index(tt, tv, npos, emb16, blkN, vv, v):
    return pl.pallas_call(
        _fused_body,
        out_shape=[
            jax.ShapeDtypeStruct((24, 8, DIM), jnp.float32),
            jax.ShapeDtypeStruct((_IDX_R, _IDX_C), jnp.int32),
        ],
    )(tt, tv, npos, emb16, blkN, vv, v)


_CG_IT1 = 110
_CG_IT2 = 110


def _cg_rows(Nmat, nrhs, iters):
    def matvec(p):
        t = jnp.dot(p, Nmat, preferred_element_type=jnp.float32)
        return p - jnp.dot(t, Nmat, preferred_element_type=jnp.float32)

    x0 = jnp.zeros_like(nrhs)
    rs0 = jnp.sum(nrhs * nrhs, axis=1, keepdims=True)

    def it(_, carry):
        x, r, p, rs = carry
        q = matvec(p)
        alpha = rs / jnp.sum(p * q, axis=1, keepdims=True)
        x = x + alpha * p
        r = r - alpha * q
        rs2 = jnp.sum(r * r, axis=1, keepdims=True)
        p = r + (rs2 / rs) * p
        return x, r, p, rs2

    x, _, _, _ = lax.fori_loop(0, iters, it, (x0, nrhs, nrhs, rs0))
    return x


def _cg_body(blkN_ref, vv_ref, v_ref, p8_ref):
    Nb = blkN_ref[...]
    N0 = blkN_ref[0:D2, 0:D2]
    vv8 = jnp.broadcast_to(vv_ref[...], (8, DIM))

    b1 = vv8 - jnp.dot(vv8, Nb, preferred_element_type=jnp.float32)
    n1 = b1 - jnp.dot(b1, Nb, preferred_element_type=jnp.float32)
    x1 = _cg_rows(Nb, n1, _CG_IT1)
    e2 = x1[0:1, 0:D2]
    e3 = x1[0:1, D2:DIM]

    rid = lax.broadcasted_iota(jnp.int32, (8, D2), 0)
    S = jnp.where(rid % 2 == 0, jnp.broadcast_to(e2, (8, D2)),
                  jnp.broadcast_to(e3, (8, D2)))
    b2 = S - jnp.dot(S, N0, preferred_element_type=jnp.float32)
    n2 = b2 - jnp.dot(b2, N0, preferred_element_type=jnp.float32)
    x2 = _cg_rows(N0, n2, _CG_IT2)

    vb = jnp.broadcast_to(v_ref[...], (8, D2))
    p8_ref[...] = jnp.where(rid < 2, vb,
                  jnp.where(rid == 2, jnp.broadcast_to(e2, (8, D2)),
                  jnp.where(rid == 3, jnp.broadcast_to(e3, (8, D2)),
                  jnp.where(rid < 6, x2, jnp.zeros((8, D2), jnp.float32)))))


def _cg_p8(blkN, vv, v):
    return pl.pallas_call(
        _cg_body,
        out_shape=jax.ShapeDtypeStruct((8, D2), jnp.float32),
    )(blkN, vv, v)


_NC = 2
_NS = 16
_NW = _NC * _NS
_BPW = NTOK // _NW
_K = 64
_NCHUNK = _BPW // _K


def _gather_body(table_hbm, idx_hbm, out_hbm, idx_v, rows_v, sem):
    wid = lax.axis_index("s") * _NC + lax.axis_index("c")
    base = wid * _BPW
    pltpu.sync_copy(idx_hbm.at[wid], idx_v)
    for ck in range(_NCHUNK):
        pltpu.async_copy(table_hbm.at[idx_v.at[ck]], rows_v, sem).wait()
        pltpu.sync_copy(rows_v, out_hbm.at[pl.ds(base + ck * _K, _K)])


@functools.cache
def _gather_rows_kernel():
    return functools.partial(
        pl.kernel,
        mesh=plsc.VectorSubcoreMesh(core_axis_name="c", subcore_axis_name="s"),
        out_type=jax.ShapeDtypeStruct((NTOK, DIM), jnp.float32),
        scratch_types=[
            pltpu.VMEM((_NCHUNK, _K), jnp.int32),
            pltpu.VMEM((_K, DIM), jnp.float32),
            pltpu.SemaphoreType.DMA,
        ],
    )(_gather_body)


def _prepare(dense_batch, embeddings, primitives_raw, identity):
    f32 = jnp.float32
    X = jnp.tril(primitives_raw.astype(f32))
    A = X - jnp.swapaxes(X, -1, -2)
    N = 0.5 * A
    zblk = jnp.zeros((D2, D2), f32)
    blkN = jnp.block([[N[0], zblk], [zblk, N[1]]])

    v = identity.astype(f32).reshape(1, D2)
    vv = jnp.concatenate([v, v], axis=1)

    emb16 = jnp.pad(embeddings.astype(f32), ((0, 5), (0, 0)))
    tt = dense_batch[0].reshape(_IDX_R, _IDX_C)
    tv = dense_batch[1].reshape(_IDX_R, _IDX_C)
    npos = dense_batch[2].reshape(_IDX_R, _IDX_C)
    return tt, tv, npos, emb16, blkN, vv, v


def kernel(dense_batch, embeddings, primitives_raw, identity):
    tt, tv, npos, emb16, blkN, vv, v = _prepare(dense_batch, embeddings,
                                                primitives_raw, identity)
    bigT3, g = _fused_table_index(tt, tv, npos, emb16, blkN, vv, v)
    bigT = bigT3.reshape(24 * 8, DIM)
    gidx = g.reshape(_NW, _NCHUNK, _K)

    out = _gather_rows_kernel()(bigT, gidx)
    return out.reshape(4, 8192, DIM)
